# Initial kernel scaffold; baseline (speedup 1.0000x reference)
import jax
import jax.numpy as jnp
from jax.experimental import pallas as pl


def kernel(x, edge_index, edge_attr, W0, b0, W1, b1, W2, b2):
    raise NotImplementedError("write your pallas kernel here")


# speedup vs baseline: 3.4960x; 3.4960x over previous
import functools
import jax
import jax.numpy as jnp
from jax import lax
from jax.experimental import pallas as pl
from jax.experimental.pallas import tpu as pltpu
from jax.experimental.pallas import tpu_sc as plsc

N = 10000
E = 320000
KS = 5
K = KS * KS
F = 128
NW = 32
EPW = E // NW
C = 80
NCHUNK = EPW // C
NP = 10240
RPT = NP // 16

@functools.cache
def _mesh():
    return plsc.VectorSubcoreMesh(
        core_axis_name="c", subcore_axis_name="s",
        num_cores=2, num_subcores=16)


def _prep_body(a0_ref, a1_ref, src_ref, basis_ref, idx_ref):
    f0 = a0_ref[...] * float(KS - 1)
    f1 = a1_ref[...] * float(KS - 1)
    fl0 = jnp.floor(f0)
    fl1 = jnp.floor(f1)
    fr0 = f0 - fl0
    fr1 = f1 - fl1
    i0 = fl0.astype(jnp.int32)
    i1 = fl1.astype(jnp.int32)
    src = src_ref[...]
    for s in range(4):
        k0 = s & 1
        k1 = (s >> 1) & 1
        b = (fr0 if k0 else (1.0 - fr0)) * (fr1 if k1 else (1.0 - fr1))
        w = jnp.clip(i0 + k0, 0, KS - 1) + KS * jnp.clip(i1 + k1, 0, KS - 1)
        basis_ref[s] = b
        idx_ref[s] = w * N + src


def _prep(a0, a1, src2d):
    nrows = a0.shape[0]
    return pl.pallas_call(
        _prep_body,
        out_shape=[
            jax.ShapeDtypeStruct((4, nrows, F), jnp.float32),
            jax.ShapeDtypeStruct((4, nrows, F), jnp.int32),
        ],
    )(a0, a1, src2d)


def _mm_body(h_ref, w_ref, y_ref):
    y_ref[0] = jnp.dot(h_ref[...], w_ref[0], preferred_element_type=jnp.float32)


def _mm(h, W):
    nblk = 1000
    return pl.pallas_call(
        _mm_body,
        grid=(K, N // nblk),
        in_specs=[
            pl.BlockSpec((nblk, F), lambda k, i: (i, 0)),
            pl.BlockSpec((1, F, F), lambda k, i: (k, 0, 0)),
        ],
        out_specs=pl.BlockSpec((1, nblk, F), lambda k, i: (k, i, 0)),
        out_shape=jax.ShapeDtypeStruct((K, N, F), jnp.float32),
    )(h, W)


def _deg_body(dst_hbm, ones_hbm, zeros_hbm, deg_hbm, dst_v, ones_v, acc):
    cid = lax.axis_index("c")
    sid = lax.axis_index("s")
    wid = cid * 16 + sid
    pltpu.sync_copy(zeros_hbm, acc.at[pl.ds(sid * RPT, RPT)])
    pltpu.sync_copy(ones_hbm, ones_v)
    plsc.subcore_barrier()

    def chunk(i, carry):
        base = wid * EPW + i * C
        pltpu.sync_copy(dst_hbm.at[pl.ds(base, C)], dst_v)
        pltpu.sync_copy(ones_v, acc.at[dst_v], add=True)
        return carry

    lax.fori_loop(0, NCHUNK, chunk, 0)
    plsc.subcore_barrier()
    pltpu.sync_copy(acc.at[pl.ds(sid * RPT, RPT)],
                    deg_hbm.at[cid, pl.ds(sid * RPT, RPT)])


@functools.cache
def _deg_call():
    return pl.kernel(
        _deg_body,
        out_type=jax.ShapeDtypeStruct((2, NP, F), jnp.float32),
        mesh=_mesh(),
        scratch_types=[
            pltpu.VMEM((C,), jnp.int32),
            pltpu.VMEM((C, F), jnp.float32),
            pltpu.VMEM_SHARED((NP, F), jnp.float32),
        ],
        compiler_params=pltpu.CompilerParams(needs_layout_passes=False),
    )


def _make_edge_body():
    def body(y_hbm, idx_hbm, basis_hbm, dst_hbm, zeros_hbm, parts_hbm,
             i0_v, i1_v, i2_v, i3_v, dst_v, basis_v,
             r0_v, r1_v, r2_v, r3_v, acc, sem):
        cid = lax.axis_index("c")
        sid = lax.axis_index("s")
        wid = cid * 16 + sid
        idx_vs = (i0_v, i1_v, i2_v, i3_v)
        rows = (r0_v, r1_v, r2_v, r3_v)

        pltpu.sync_copy(zeros_hbm, acc.at[pl.ds(sid * RPT, RPT)])
        plsc.subcore_barrier()

        def chunk(i, carry):
            base = wid * EPW + i * C
            pltpu.sync_copy(dst_hbm.at[pl.ds(base, C)], dst_v)
            for s in range(4):
                pltpu.sync_copy(basis_hbm.at[pl.ds(s * E + base, C)],
                                basis_v.at[pl.ds(s * C, C)])
                pltpu.sync_copy(idx_hbm.at[pl.ds(s * E + base, C)], idx_vs[s])
            cps = [pltpu.async_copy(y_hbm.at[idx_vs[s]], rows[s], sem)
                   for s in range(4)]
            for cp in cps:
                cp.wait()

            def group(g, carry2):
                for j in range(16):
                    c = g * 16 + j
                    bb = [plsc.load_gather(
                              basis_v, [jnp.full((16,), s * C, jnp.int32) + c])
                          for s in range(4)]
                    for kk in range(8):
                        a = bb[0] * rows[0][c, pl.ds(kk * 16, 16)]
                        a = a + bb[1] * rows[1][c, pl.ds(kk * 16, 16)]
                        a = a + bb[2] * rows[2][c, pl.ds(kk * 16, 16)]
                        a = a + bb[3] * rows[3][c, pl.ds(kk * 16, 16)]
                        rows[0][c, pl.ds(kk * 16, 16)] = a
                return carry2

            lax.fori_loop(0, C // 16, group, 0)
            pltpu.sync_copy(r0_v, acc.at[dst_v], add=True)
            return carry

        lax.fori_loop(0, NCHUNK, chunk, 0)
        plsc.subcore_barrier()
        pltpu.sync_copy(acc.at[pl.ds(sid * RPT, RPT)],
                        parts_hbm.at[cid, pl.ds(sid * RPT, RPT)])

    return body


@functools.cache
def _edge_call():
    return pl.kernel(
        _make_edge_body(),
        out_type=jax.ShapeDtypeStruct((2, NP, F), jnp.float32),
        mesh=_mesh(),
        scratch_types=[
        pltpu.VMEM((C,), jnp.int32),
        pltpu.VMEM((C,), jnp.int32),
        pltpu.VMEM((C,), jnp.int32),
        pltpu.VMEM((C,), jnp.int32),
        pltpu.VMEM((C,), jnp.int32),
        pltpu.VMEM((4 * C,), jnp.float32),
        pltpu.VMEM((C, F), jnp.float32),
        pltpu.VMEM((C, F), jnp.float32),
        pltpu.VMEM((C, F), jnp.float32),
        pltpu.VMEM((C, F), jnp.float32),
            pltpu.VMEM_SHARED((NP, F), jnp.float32),
            pltpu.SemaphoreType.DMA,
        ],
        compiler_params=pltpu.CompilerParams(needs_layout_passes=False),
    )


def _post_body(p_ref, d_ref, b_ref, y_ref, h_ref):
    out = p_ref[0] + p_ref[1]
    deg = d_ref[0, :, 0:1] + d_ref[1, :, 0:1]
    out = out / jnp.maximum(deg, 1.0) + b_ref[0]
    n = jnp.sqrt(jnp.sum(out * out, axis=1, keepdims=True))
    y_ref[...] = out / jnp.maximum(n, 1e-12)
    h_ref[...] = jnp.maximum(out, 0.0)


def _post(parts, degp, b2d):
    blk = 1000
    return pl.pallas_call(
        _post_body,
        grid=(N // blk,),
        in_specs=[
            pl.BlockSpec((2, blk, F), lambda i: (0, i, 0)),
            pl.BlockSpec((2, blk, F), lambda i: (0, i, 0)),
            pl.BlockSpec((1, F), lambda i: (0, 0)),
        ],
        out_specs=[
            pl.BlockSpec((blk, F), lambda i: (i, 0)),
            pl.BlockSpec((blk, F), lambda i: (i, 0)),
        ],
        out_shape=[
            jax.ShapeDtypeStruct((N, F), jnp.float32),
            jax.ShapeDtypeStruct((N, F), jnp.float32),
        ],
    )(parts, degp, b2d)


def _xnorm_body(x_ref, y_ref):
    x = x_ref[...]
    n = jnp.sqrt(jnp.sum(x * x, axis=1, keepdims=True))
    y_ref[...] = x / jnp.maximum(n, 1e-12)


def _xnorm(x):
    blk = 1000
    return pl.pallas_call(
        _xnorm_body,
        grid=(N // blk,),
        in_specs=[pl.BlockSpec((blk, F), lambda i: (i, 0))],
        out_specs=pl.BlockSpec((blk, F), lambda i: (i, 0)),
        out_shape=jax.ShapeDtypeStruct((N, F), jnp.float32),
    )(x)


def kernel(x, edge_index, edge_attr, W0, b0, W1, b1, W2, b2):
    src = edge_index[0].astype(jnp.int32)
    dst = edge_index[1].astype(jnp.int32)
    a0 = edge_attr[:, 0].reshape(E // F, F)
    a1 = edge_attr[:, 1].reshape(E // F, F)
    src2d = src.reshape(E // F, F)

    basis3, idx3 = _prep(a0, a1, src2d)
    basis = basis3.reshape(4 * E)
    idx = idx3.reshape(4 * E)

    onesF = jnp.ones((C, F), jnp.float32)
    zerosF = jnp.zeros((RPT, F), jnp.float32)

    degp = _deg_call()(dst, onesF, zerosF)

    xs = [_xnorm(x)]
    h = x
    for (W, b) in ((W0, b0), (W1, b1), (W2, b2)):
        Y = _mm(h, W).reshape(K * N, F)
        parts = _edge_call()(Y, idx, basis, dst, zerosF)
        y_norm, h = _post(parts, degp, b.reshape(1, F))
        xs.append(y_norm)
    return jnp.concatenate(xs, axis=-1)
